# baseline (speedup 1.0000x reference)
import jax
import jax.numpy as jnp
from jax.experimental import pallas as pl
from jax.experimental.pallas import tpu as pltpu

_EMBED = 1024
_HEADS = 16
_HDIM = 64
_SEQ = 2048
_QCHUNK = 512
_PAIRS = _HEADS // 2
_PCHUNK = 128
_PSTEPS = _SEQ // _PCHUNK
_WCHUNK = 512
_WSTEPS = 3 * _EMBED // _WCHUNK
_EXP2_SCALE = 1.4426950408889634 / 8.0
_EXP2_CLAMP = 120.0


def _mha_body(xq_ref, xk_ref, xv_ref, w3_ref, b3_ref, wo_ref, bo_ref,
              out_ref, w3s_ref, qkv_ref, acc_ref):
    j = pl.program_id(0)
    f32 = jnp.float32
    bf16 = jnp.bfloat16
    dn = (((1,), (1,)), ((), ()))

    @pl.when(j < _WSTEPS)
    def _wcast():
        w3s_ref[pl.ds(j * _WCHUNK, _WCHUNK), :] = w3_ref[...].astype(bf16)

    @pl.when(jnp.logical_and(j >= _WSTEPS, j < _WSTEPS + _PSTEPS))
    def _proj():
        row0 = (j - _WSTEPS) * _PCHUNK
        for t, x_ref in enumerate((xq_ref, xk_ref, xv_ref)):
            xb = x_ref[...].astype(bf16)
            w_t = w3s_ref[t * _EMBED:(t + 1) * _EMBED]
            p = jax.lax.dot_general(xb, w_t, dn, preferred_element_type=f32)
            p = p + b3_ref[t:t + 1]
            qkv_ref[pl.ds(row0, _PCHUNK), t * _EMBED:(t + 1) * _EMBED] = (
                p.astype(bf16))

    @pl.when(jnp.logical_and(j >= _WSTEPS + _PSTEPS,
                             j < _WSTEPS + _PSTEPS + _PAIRS))
    def _heads():
        ones_col = (jax.lax.broadcasted_iota(jnp.int32, (_SEQ, _HDIM), 1) == 0)
        jp = j - (_WSTEPS + _PSTEPS)
        lane0 = jp * 2 * _HDIM
        q_pair = qkv_ref[:, pl.ds(lane0, 2 * _HDIM)]
        k_pair = qkv_ref[:, pl.ds(_EMBED + lane0, 2 * _HDIM)]
        v_pair = qkv_ref[:, pl.ds(2 * _EMBED + lane0, 2 * _HDIM)]
        for hh in range(2):
            sl_h = slice(hh * _HDIM, (hh + 1) * _HDIM)
            q_h = q_pair[:, sl_h]
            k_h = k_pair[:, sl_h]
            v_ext = jnp.concatenate([v_pair[:, sl_h], ones_col.astype(bf16)],
                                    axis=1)
            for i in range(_SEQ // _QCHUNK):
                qc = q_h[i * _QCHUNK:(i + 1) * _QCHUNK]
                s = jax.lax.dot_general(qc, k_h, dn, preferred_element_type=f32)
                e = jnp.exp2(jnp.minimum(s * _EXP2_SCALE, _EXP2_CLAMP)).astype(bf16)
                o_ext = jnp.dot(e, v_ext, preferred_element_type=f32)
                r = o_ext[:, _HDIM:_HDIM + 1]
                acc_ref[pl.ds(i * _QCHUNK, _QCHUNK),
                        pl.ds(2 * _HDIM * (2 * jp + hh), 2 * _HDIM)] = (
                    (o_ext / r).astype(bf16))

    @pl.when(j == _WSTEPS + _PSTEPS + _PAIRS)
    def _outproj():
        out_ref[...] = jax.lax.dot_general(
            acc_ref[...], wo_ref[...], dn,
            preferred_element_type=f32) + bo_ref[...]


def kernel(query, key, value, in_proj_weight, in_proj_bias,
           ln_q_g, ln_q_b, gp_q_w, gp_q_b,
           ln_k_g, ln_k_b, gp_k_w, gp_k_b,
           ln_v_g, ln_v_b, gp_v_w, gp_v_b,
           out_w, out_b):
    del ln_q_g, ln_q_b, gp_q_w, gp_q_b, ln_k_g, ln_k_b, gp_k_w, gp_k_b
    del ln_v_g, ln_v_b, gp_v_w, gp_v_b
    bf16 = jnp.bfloat16
    xq = query[:, 0, :]
    xk = key[:, 0, :]
    xv = value[:, 0, :]
    b3 = in_proj_bias.reshape(3, _EMBED)
    bo = out_b.reshape(1, _EMBED)
    wo_big = jnp.pad(out_w.astype(bf16).reshape(_EMBED, _HEADS, _HDIM),
                     ((0, 0), (0, 0), (0, _HDIM))).reshape(_EMBED, 2 * _EMBED)

    def xmap(j):
        return (jnp.clip(j - _WSTEPS, 0, _PSTEPS - 1), 0)

    def wmap(j):
        return (jnp.minimum(j, _WSTEPS - 1), 0)

    out2d = pl.pallas_call(
        _mha_body,
        grid=(_WSTEPS + _PSTEPS + _PAIRS + 1,),
        in_specs=[
            pl.BlockSpec((_PCHUNK, _EMBED), xmap),
            pl.BlockSpec((_PCHUNK, _EMBED), xmap),
            pl.BlockSpec((_PCHUNK, _EMBED), xmap),
            pl.BlockSpec((_WCHUNK, _EMBED), wmap),
            pl.BlockSpec((3, _EMBED), lambda j: (0, 0)),
            pl.BlockSpec((_EMBED, 2 * _EMBED), lambda j: (0, 0)),
            pl.BlockSpec((1, _EMBED), lambda j: (0, 0)),
        ],
        out_specs=pl.BlockSpec((_SEQ, _EMBED), lambda j: (0, 0)),
        out_shape=jax.ShapeDtypeStruct((_SEQ, _EMBED), jnp.float32),
        scratch_shapes=[pltpu.VMEM((3 * _EMBED, _EMBED), bf16),
                        pltpu.VMEM((_SEQ, 3 * _EMBED), bf16),
                        pltpu.VMEM((_SEQ, 2 * _EMBED), bf16)],
    )(xq, xk, xv, in_proj_weight, b3, wo_big, bo)
    return out2d[:, None, :]


# speedup vs baseline: 1.1056x; 1.1056x over previous
import jax
import jax.numpy as jnp
from jax.experimental import pallas as pl
from jax.experimental.pallas import tpu as pltpu

_EMBED = 1024
_HEADS = 16
_HDIM = 64
_SEQ = 2048
_QCHUNK = 256
_PCHUNK = 512
_EXP2_SCALE = 1.4426950408889634 / 8.0
_EXP2_CLAMP = 120.0


def _mha_body(xq_ref, xk_ref, xv_ref, w3_ref, b3_ref, wo_ref, bo_ref,
              out_ref, qkv_ref, acc_ref, xbuf_ref, sem_ref):
    j = pl.program_id(0)
    f32 = jnp.float32
    bf16 = jnp.bfloat16
    dn = (((1,), (1,)), ((), ()))

    @pl.when(j == 0)
    def _proj():
        xs = (xq_ref, xk_ref, xv_ref)
        n = (_SEQ // _PCHUNK) * 3

        def copy(idx):
            c, t = divmod(idx, 3)
            return pltpu.make_async_copy(
                xs[t].at[pl.ds(c * _PCHUNK, _PCHUNK), :],
                xbuf_ref.at[idx % 2], sem_ref.at[idx % 2])

        copy(0).start()
        for idx in range(n):
            c, t = divmod(idx, 3)
            if idx + 1 < n:
                copy(idx + 1).start()
            copy(idx).wait()
            xb = xbuf_ref[idx % 2].astype(bf16)
            w_t = w3_ref[t * _EMBED:(t + 1) * _EMBED]
            p = jax.lax.dot_general(xb, w_t, dn, preferred_element_type=f32)
            p = p + b3_ref[t:t + 1]
            qkv_ref[pl.ds(c * _PCHUNK, _PCHUNK),
                    t * _EMBED:(t + 1) * _EMBED] = p.astype(bf16)

    @pl.when(j == 1)
    def _heads():
        ones_col = (jax.lax.broadcasted_iota(jnp.int32, (_SEQ, _HDIM), 1) == 0)

        def pair_body(pp, carry):
            lane0 = pp * 2 * _HDIM
            q_pair = qkv_ref[:, pl.ds(lane0, 2 * _HDIM)]
            k_pair = qkv_ref[:, pl.ds(_EMBED + lane0, 2 * _HDIM)]
            v_pair = qkv_ref[:, pl.ds(2 * _EMBED + lane0, 2 * _HDIM)]
            for hh in range(2):
                sl_h = slice(hh * _HDIM, (hh + 1) * _HDIM)
                q_h = q_pair[:, sl_h]
                k_h = k_pair[:, sl_h]
                v_ext = jnp.concatenate(
                    [v_pair[:, sl_h], ones_col.astype(bf16)], axis=1)
                for i in range(_SEQ // _QCHUNK):
                    qc = q_h[i * _QCHUNK:(i + 1) * _QCHUNK]
                    s = jax.lax.dot_general(qc, k_h, dn,
                                            preferred_element_type=f32)
                    e = jnp.exp2(jnp.minimum(s * _EXP2_SCALE,
                                             _EXP2_CLAMP)).astype(bf16)
                    o_ext = jnp.dot(e, v_ext, preferred_element_type=f32)
                    r = o_ext[:, _HDIM:_HDIM + 1]
                    acc_ref[pl.ds(i * _QCHUNK, _QCHUNK),
                            pl.ds(2 * lane0 + hh * 2 * _HDIM, 2 * _HDIM)] = (
                        (o_ext / r).astype(bf16))
            return carry

        jax.lax.fori_loop(0, _HEADS // 2, pair_body, 0)

    @pl.when(j == 2)
    def _outproj():
        for c in range(_SEQ // _PCHUNK):
            rows = pl.ds(c * _PCHUNK, _PCHUNK)
            out_ref[rows, :] = jnp.dot(acc_ref[rows, :], wo_ref[...],
                                       preferred_element_type=f32) + bo_ref[...]


def kernel(query, key, value, in_proj_weight, in_proj_bias,
           ln_q_g, ln_q_b, gp_q_w, gp_q_b,
           ln_k_g, ln_k_b, gp_k_w, gp_k_b,
           ln_v_g, ln_v_b, gp_v_w, gp_v_b,
           out_w, out_b):
    del ln_q_g, ln_q_b, gp_q_w, gp_q_b, ln_k_g, ln_k_b, gp_k_w, gp_k_b
    del ln_v_g, ln_v_b, gp_v_w, gp_v_b
    bf16 = jnp.bfloat16
    xq = query[:, 0, :]
    xk = key[:, 0, :]
    xv = value[:, 0, :]
    w3 = in_proj_weight.astype(bf16)
    b3 = in_proj_bias.reshape(3, _EMBED)
    bo = out_b.reshape(1, _EMBED)
    wo_big = jnp.pad(out_w.T.astype(bf16).reshape(_HEADS, _HDIM, _EMBED),
                     ((0, 0), (0, _HDIM), (0, 0))).reshape(2 * _EMBED, _EMBED)
    out2d = pl.pallas_call(
        _mha_body,
        grid=(3,),
        in_specs=[
            pl.BlockSpec(memory_space=pl.ANY),
            pl.BlockSpec(memory_space=pl.ANY),
            pl.BlockSpec(memory_space=pl.ANY),
            pl.BlockSpec((3 * _EMBED, _EMBED), lambda j: (0, 0)),
            pl.BlockSpec((3, _EMBED), lambda j: (0, 0)),
            pl.BlockSpec((2 * _EMBED, _EMBED), lambda j: (0, 0)),
            pl.BlockSpec((1, _EMBED), lambda j: (0, 0)),
        ],
        out_specs=pl.BlockSpec((_SEQ, _EMBED), lambda j: (0, 0)),
        out_shape=jax.ShapeDtypeStruct((_SEQ, _EMBED), jnp.float32),
        scratch_shapes=[pltpu.VMEM((_SEQ, 3 * _EMBED), bf16),
                        pltpu.VMEM((_SEQ, 2 * _EMBED), bf16),
                        pltpu.VMEM((2, _PCHUNK, _EMBED), jnp.float32),
                        pltpu.SemaphoreType.DMA((2,))],
    )(xq, xk, xv, w3, b3, wo_big, bo)
    return out2d[:, None, :]
